# baseline (speedup 1.0000x reference)
import functools

import jax
import jax.numpy as jnp
from jax import lax
from jax.experimental import pallas as pl
from jax.experimental.pallas import tpu as pltpu
from jax.experimental.pallas import tpu_sc as plsc

B, F, V, E = 4096, 26, 1000000, 16
BF = B * F
FE = F * E
H = 400
EPS = 1e-3

NC, NS = 2, 16
NW = NC * NS
B_PER_W = BF // NW
CHUNK = 128
NCH = B_PER_W // CHUNK
PAD_NCH = 32
NSTR = E + 1
BLK = 512
assert B_PER_W % CHUNK == 0 and B % BLK == 0


def _sc_gather(idx_all, embt_flat, fw_flat):
    mesh = plsc.VectorSubcoreMesh(core_axis_name="c", subcore_axis_name="s")

    @functools.partial(
        pl.kernel,
        out_type=jax.ShapeDtypeStruct((NSTR * BF,), jnp.float32),
        mesh=mesh,
        scratch_types=[
            pltpu.VMEM((PAD_NCH, CHUNK), jnp.int32),
            pltpu.VMEM((B_PER_W,), jnp.float32),
            pltpu.SemaphoreType.DMA,
        ],
    )
    def sc_kernel(idx_hbm, embt_hbm, fw_hbm, out_hbm, idx_v, buf, sem):
        wid = lax.axis_index("s") * NC + lax.axis_index("c")
        base = wid * B_PER_W

        for s in range(NSTR):
            table = embt_hbm if s < E else fw_hbm
            row0 = (s * NW + wid) * PAD_NCH
            pltpu.sync_copy(idx_hbm.at[pl.ds(row0, PAD_NCH)], idx_v)

            @pl.loop(0, NCH)
            def _(j):
                pltpu.async_copy(table.at[idx_v.at[j]],
                                 buf.at[pl.ds(j * CHUNK, CHUNK)], sem).wait()

            pltpu.sync_copy(buf, out_hbm.at[pl.ds(s * BF + base, B_PER_W)])

    return sc_kernel(idx_all, embt_flat, fw_flat)


def _mlp_body(fev_ref, yfw_ref, fv_ref, r26_ref, d1_ref, b1_ref, d2_ref,
              b2_ref, d3_ref, b3_ref, s_ref, w1_ref, w2_ref, w3_ref,
              bias_ref, out_ref):
    f32 = jnp.float32
    hi = lax.Precision.HIGHEST
    fv = fv_ref[...]
    fv_rep = lax.dot_general(fv, r26_ref[...], (((1,), (0,)), ((), ())),
                             precision=hi, preferred_element_type=f32)
    fev = fev_ref[...] * fv_rep

    acc = lax.dot_general(yfw_ref[...] * fv, w1_ref[...],
                          (((1,), (0,)), ((), ())),
                          precision=hi, preferred_element_type=f32)

    summed = lax.dot_general(fev, s_ref[...], (((1,), (0,)), ((), ())),
                             precision=hi, preferred_element_type=f32)
    part2 = lax.dot_general(fev * fev, s_ref[...], (((1,), (0,)), ((), ())),
                            precision=hi, preferred_element_type=f32)
    y2 = 0.5 * (summed * summed - part2)
    acc += lax.dot_general(y2, w2_ref[...], (((1,), (0,)), ((), ())),
                           precision=hi, preferred_element_type=f32)

    h = lax.dot_general(fev, d1_ref[...], (((1,), (0,)), ((), ())),
                        precision=hi, preferred_element_type=f32)
    h = jnp.maximum(h + b1_ref[...], 0.0)
    h = lax.dot_general(h, d2_ref[...], (((1,), (0,)), ((), ())),
                        precision=hi, preferred_element_type=f32)
    h = jnp.maximum(h + b2_ref[...], 0.0)
    h = lax.dot_general(h, d3_ref[...], (((1,), (0,)), ((), ())),
                        precision=hi, preferred_element_type=f32)
    h = jnp.maximum(h + b3_ref[...], 0.0)
    acc += lax.dot_general(h, w3_ref[...], (((1,), (0,)), ((), ())),
                           precision=hi, preferred_element_type=f32)
    out_ref[...] = acc + bias_ref[...]


def kernel(feat_index, feat_value, first_w, emb, d1_k, d1_b, bn1_g, bn1_b,
           d2_k, d2_b, bn2_g, bn2_b, d3_k, d3_b, bn3_g, bn3_b, fc_k, fc_b):
    f32 = jnp.float32
    i32 = jnp.int32
    idx = feat_index.reshape(NW, NCH, CHUNK).astype(i32)
    idx = jnp.pad(idx, ((0, 0), (0, PAD_NCH - NCH), (0, 0)))
    offs = jnp.concatenate([jnp.arange(E, dtype=i32) * V,
                            jnp.zeros((1,), i32)])
    idx_all = (idx[None, :, :, :] + offs[:, None, None, None]).reshape(
        NSTR * NW * PAD_NCH, CHUNK)

    embt_flat = emb.T.reshape(-1)
    fw_flat = first_w.reshape(-1)

    gath = _sc_gather(idx_all, embt_flat, fw_flat).reshape(NSTR, BF)

    fev_raw = gath[0:E].T.reshape(B, FE)
    yfw_raw = gath[E].reshape(B, F)
    fv = feat_value

    r26 = (lax.broadcasted_iota(i32, (F, FE), 0) ==
           lax.broadcasted_iota(i32, (F, FE), 1) // E).astype(f32)

    inv = 1.0 / jnp.sqrt(1.0 + EPS)
    d1 = d1_k * (bn1_g * inv)[None, :]
    b1 = (d1_b * bn1_g * inv + bn1_b)[None, :]
    d2 = d2_k * (bn2_g * inv)[None, :]
    b2 = (d2_b * bn2_g * inv + bn2_b)[None, :]
    d3 = d3_k * (bn3_g * inv)[None, :]
    b3 = (d3_b * bn3_g * inv + bn3_b)[None, :]

    w1 = fc_k[0:F, :]
    w2 = fc_k[F:F + E, :]
    w3 = fc_k[F + E:, :]
    bias = fc_b[None, :]

    s = (lax.broadcasted_iota(i32, (FE, E), 0) % E ==
         lax.broadcasted_iota(i32, (FE, E), 1)).astype(f32)

    grid = (B // BLK,)
    bspec = lambda w: pl.BlockSpec((BLK, w), lambda i: (i, 0))
    wspec = lambda shp: pl.BlockSpec(shp, lambda i: (0, 0))

    out = pl.pallas_call(
        _mlp_body,
        grid=grid,
        in_specs=[
            bspec(FE), bspec(F), bspec(F),
            wspec((F, FE)),
            wspec((FE, H)), wspec((1, H)),
            wspec((H, H)), wspec((1, H)),
            wspec((H, H)), wspec((1, H)),
            wspec((FE, E)),
            wspec((F, 1)), wspec((E, 1)), wspec((H, 1)), wspec((1, 1)),
        ],
        out_specs=pl.BlockSpec((BLK, 1), lambda i: (i, 0)),
        out_shape=jax.ShapeDtypeStruct((B, 1), f32),
    )(fev_raw, yfw_raw, fv, r26, d1, b1, d2, b2, d3, b3, s, w1, w2, w3, bias)
    return out


# speedup vs baseline: 1.1604x; 1.1604x over previous
import functools

import jax
import jax.numpy as jnp
from jax import lax
from jax.experimental import pallas as pl
from jax.experimental.pallas import tpu as pltpu
from jax.experimental.pallas import tpu_sc as plsc

B, F, V, E = 4096, 26, 1000000, 16
BF = B * F
FE = F * E
H = 400
EPS = 1e-3

NC, NS = 2, 16
NW = NC * NS
B_PER_W = BF // NW
CHUNK = 128
NCH = B_PER_W // CHUNK
PAD_NCH = 32
NSTR = E + 1
FCH = 4096
NFCH = V // FCH
FTAIL = V - NFCH * FCH
BLK = 512
assert B_PER_W % CHUNK == 0 and B % BLK == 0 and FTAIL % 8 == 0


def _sc_flatten(embT):
    mesh = plsc.VectorSubcoreMesh(core_axis_name="c", subcore_axis_name="s")

    @functools.partial(
        pl.kernel,
        out_type=jax.ShapeDtypeStruct((E * V,), jnp.float32),
        mesh=mesh,
        scratch_types=[
            pltpu.VMEM((E, FCH), jnp.float32),
            pltpu.VMEM((E, FTAIL), jnp.float32),
            pltpu.SemaphoreType.DMA,
        ],
    )
    def flat_kernel(embt_hbm, out_hbm, buf, tail, sem):
        wid = lax.axis_index("s") * NC + lax.axis_index("c")

        @pl.loop(0, (NFCH + NW - 1) // NW)
        def _(k):
            c = k * NW + wid

            @pl.when(c < NFCH)
            def _():
                col0 = c * FCH
                pltpu.sync_copy(embt_hbm.at[:, pl.ds(col0, FCH)], buf)

                for e in range(E):
                    pltpu.async_copy(buf.at[e],
                                     out_hbm.at[pl.ds(e * V + col0, FCH)],
                                     sem)
                for e in range(E):
                    pltpu.make_async_copy(
                        buf.at[e],
                        out_hbm.at[pl.ds(e * V + c * FCH, FCH)], sem).wait()

        @pl.when(wid == 0)
        def _():
            t0 = NFCH * FCH
            pltpu.sync_copy(embt_hbm.at[:, pl.ds(t0, FTAIL)], tail)
            for e in range(E):
                pltpu.sync_copy(tail.at[e],
                                out_hbm.at[pl.ds(e * V + t0, FTAIL)])

    return flat_kernel(embT)


def _sc_gather(idx_all, embt_flat, fw_flat):
    mesh = plsc.VectorSubcoreMesh(core_axis_name="c", subcore_axis_name="s")

    @functools.partial(
        pl.kernel,
        out_type=jax.ShapeDtypeStruct((NSTR * BF,), jnp.float32),
        mesh=mesh,
        scratch_types=[
            pltpu.VMEM((PAD_NCH, CHUNK), jnp.int32),
            pltpu.VMEM((2, B_PER_W), jnp.float32),
            pltpu.SemaphoreType.DMA,
            pltpu.SemaphoreType.DMA,
        ],
    )
    def sc_kernel(idx_hbm, embt_hbm, fw_hbm, out_hbm, idx_v, buf, sem, wsem):
        wid = lax.axis_index("s") * NC + lax.axis_index("c")
        base = wid * B_PER_W

        for s in range(NSTR):
            table = embt_hbm if s < E else fw_hbm
            row0 = (s * NW + wid) * PAD_NCH
            pltpu.sync_copy(idx_hbm.at[pl.ds(row0, PAD_NCH)], idx_v)
            bsel = s % 2

            if s >= 2:
                pltpu.make_async_copy(
                    buf.at[s % 2],
                    out_hbm.at[pl.ds((s - 2) * BF + base, B_PER_W)],
                    wsem).wait()

            @pl.loop(0, NCH)
            def _(j):
                pltpu.async_copy(table.at[idx_v.at[j]],
                                 buf.at[bsel, pl.ds(j * CHUNK, CHUNK)], sem)

            @pl.loop(0, NCH)
            def _(j):
                pltpu.make_async_copy(
                    table.at[idx_v.at[j]],
                    buf.at[bsel, pl.ds(j * CHUNK, CHUNK)], sem).wait()

            pltpu.async_copy(buf.at[bsel],
                             out_hbm.at[pl.ds(s * BF + base, B_PER_W)], wsem)

        for s in (NSTR - 2, NSTR - 1):
            pltpu.make_async_copy(
                buf.at[s % 2],
                out_hbm.at[pl.ds(s * BF + base, B_PER_W)], wsem).wait()

    return sc_kernel(idx_all, embt_flat, fw_flat)


def _mlp_body(fev_ref, yfw_ref, fv_ref, r26_ref, d1_ref, b1_ref, d2_ref,
              b2_ref, d3_ref, b3_ref, s_ref, w1_ref, w2_ref, w3_ref,
              bias_ref, out_ref):
    f32 = jnp.float32
    hi = lax.Precision.HIGHEST
    fv = fv_ref[...]
    fv_rep = lax.dot_general(fv, r26_ref[...], (((1,), (0,)), ((), ())),
                             precision=hi, preferred_element_type=f32)
    fev = fev_ref[...] * fv_rep

    acc = lax.dot_general(yfw_ref[...] * fv, w1_ref[...],
                          (((1,), (0,)), ((), ())),
                          precision=hi, preferred_element_type=f32)

    summed = lax.dot_general(fev, s_ref[...], (((1,), (0,)), ((), ())),
                             precision=hi, preferred_element_type=f32)
    part2 = lax.dot_general(fev * fev, s_ref[...], (((1,), (0,)), ((), ())),
                            precision=hi, preferred_element_type=f32)
    y2 = 0.5 * (summed * summed - part2)
    acc += lax.dot_general(y2, w2_ref[...], (((1,), (0,)), ((), ())),
                           precision=hi, preferred_element_type=f32)

    h = lax.dot_general(fev, d1_ref[...], (((1,), (0,)), ((), ())),
                        precision=hi, preferred_element_type=f32)
    h = jnp.maximum(h + b1_ref[...], 0.0)
    h = lax.dot_general(h, d2_ref[...], (((1,), (0,)), ((), ())),
                        precision=hi, preferred_element_type=f32)
    h = jnp.maximum(h + b2_ref[...], 0.0)
    h = lax.dot_general(h, d3_ref[...], (((1,), (0,)), ((), ())),
                        precision=hi, preferred_element_type=f32)
    h = jnp.maximum(h + b3_ref[...], 0.0)
    acc += lax.dot_general(h, w3_ref[...], (((1,), (0,)), ((), ())),
                           precision=hi, preferred_element_type=f32)
    out_ref[...] = acc + bias_ref[...]


def kernel(feat_index, feat_value, first_w, emb, d1_k, d1_b, bn1_g, bn1_b,
           d2_k, d2_b, bn2_g, bn2_b, d3_k, d3_b, bn3_g, bn3_b, fc_k, fc_b):
    f32 = jnp.float32
    i32 = jnp.int32
    idx = feat_index.reshape(NW, NCH, CHUNK).astype(i32)
    idx = jnp.pad(idx, ((0, 0), (0, PAD_NCH - NCH), (0, 0)))
    offs = jnp.concatenate([jnp.arange(E, dtype=i32) * V,
                            jnp.zeros((1,), i32)])
    idx_all = (idx[None, :, :, :] + offs[:, None, None, None]).reshape(
        NSTR * NW * PAD_NCH, CHUNK)

    embt_flat = lax.reshape(emb, (E * V,), dimensions=(1, 0))
    fw_flat = first_w.reshape(-1)

    gath = _sc_gather(idx_all, embt_flat, fw_flat).reshape(NSTR, BF)

    fev_raw = gath[0:E].T.reshape(B, FE)
    yfw_raw = gath[E].reshape(B, F)
    fv = feat_value

    r26 = (lax.broadcasted_iota(i32, (F, FE), 0) ==
           lax.broadcasted_iota(i32, (F, FE), 1) // E).astype(f32)

    inv = 1.0 / jnp.sqrt(1.0 + EPS)
    d1 = d1_k * (bn1_g * inv)[None, :]
    b1 = (d1_b * bn1_g * inv + bn1_b)[None, :]
    d2 = d2_k * (bn2_g * inv)[None, :]
    b2 = (d2_b * bn2_g * inv + bn2_b)[None, :]
    d3 = d3_k * (bn3_g * inv)[None, :]
    b3 = (d3_b * bn3_g * inv + bn3_b)[None, :]

    w1 = fc_k[0:F, :]
    w2 = fc_k[F:F + E, :]
    w3 = fc_k[F + E:, :]
    bias = fc_b[None, :]

    s = (lax.broadcasted_iota(i32, (FE, E), 0) % E ==
         lax.broadcasted_iota(i32, (FE, E), 1)).astype(f32)

    grid = (B // BLK,)
    bspec = lambda w: pl.BlockSpec((BLK, w), lambda i: (i, 0))
    wspec = lambda shp: pl.BlockSpec(shp, lambda i: (0, 0))

    out = pl.pallas_call(
        _mlp_body,
        grid=grid,
        in_specs=[
            bspec(FE), bspec(F), bspec(F),
            wspec((F, FE)),
            wspec((FE, H)), wspec((1, H)),
            wspec((H, H)), wspec((1, H)),
            wspec((H, H)), wspec((1, H)),
            wspec((FE, E)),
            wspec((F, 1)), wspec((E, 1)), wspec((H, 1)), wspec((1, 1)),
        ],
        out_specs=pl.BlockSpec((BLK, 1), lambda i: (i, 0)),
        out_shape=jax.ShapeDtypeStruct((B, 1), f32),
    )(fev_raw, yfw_raw, fv, r26, d1, b1, d2, b2, d3, b3, s, w1, w2, w3, bias)
    return out


# speedup vs baseline: 1.5791x; 1.3608x over previous
import functools

import jax
import jax.numpy as jnp
from jax import lax
from jax.experimental import pallas as pl
from jax.experimental.pallas import tpu as pltpu
from jax.experimental.pallas import tpu_sc as plsc

B, F, V, E = 4096, 26, 1000000, 16
BF = B * F
FE = F * E
H = 400
EPS = 1e-3

NC, NS = 2, 16
NW = NC * NS
B_PER_W = BF // NW
CHUNK = 128
NCH = B_PER_W // CHUNK
PAD_NCH = 32
NSTR = E + 1
FCH = 4096
NFCH = V // FCH
FTAIL = V - NFCH * FCH
BLK = 512
assert B_PER_W % CHUNK == 0 and B % BLK == 0 and FTAIL % 8 == 0


def _sc_flatten(embT):
    mesh = plsc.VectorSubcoreMesh(core_axis_name="c", subcore_axis_name="s")

    @functools.partial(
        pl.kernel,
        out_type=jax.ShapeDtypeStruct((E * V,), jnp.float32),
        mesh=mesh,
        scratch_types=[
            pltpu.VMEM((E, FCH), jnp.float32),
            pltpu.VMEM((E, FTAIL), jnp.float32),
            pltpu.SemaphoreType.DMA,
        ],
    )
    def flat_kernel(embt_hbm, out_hbm, buf, tail, sem):
        wid = lax.axis_index("s") * NC + lax.axis_index("c")

        @pl.loop(0, (NFCH + NW - 1) // NW)
        def _(k):
            c = k * NW + wid

            @pl.when(c < NFCH)
            def _():
                col0 = c * FCH
                pltpu.sync_copy(embt_hbm.at[:, pl.ds(col0, FCH)], buf)

                for e in range(E):
                    pltpu.async_copy(buf.at[e],
                                     out_hbm.at[pl.ds(e * V + col0, FCH)],
                                     sem)
                for e in range(E):
                    pltpu.make_async_copy(
                        buf.at[e],
                        out_hbm.at[pl.ds(e * V + c * FCH, FCH)], sem).wait()

        @pl.when(wid == 0)
        def _():
            t0 = NFCH * FCH
            pltpu.sync_copy(embt_hbm.at[:, pl.ds(t0, FTAIL)], tail)
            for e in range(E):
                pltpu.sync_copy(tail.at[e],
                                out_hbm.at[pl.ds(e * V + t0, FTAIL)])

    return flat_kernel(embT)


def _sc_gather(idx_all, embt_flat, fw_flat):
    mesh = plsc.VectorSubcoreMesh(core_axis_name="c", subcore_axis_name="s")

    @functools.partial(
        pl.kernel,
        out_type=jax.ShapeDtypeStruct((NSTR * BF,), jnp.float32),
        mesh=mesh,
        scratch_types=[
            pltpu.VMEM((PAD_NCH, CHUNK), jnp.int32),
            pltpu.VMEM((2, B_PER_W), jnp.float32),
            pltpu.SemaphoreType.DMA,
            pltpu.SemaphoreType.DMA,
        ],
    )
    def sc_kernel(idx_hbm, embt_hbm, fw_hbm, out_hbm, idx_v, buf, sem, wsem):
        wid = lax.axis_index("s") * NC + lax.axis_index("c")
        base = wid * B_PER_W

        for s in range(NSTR):
            table = embt_hbm if s < E else fw_hbm
            row0 = (s * NW + wid) * PAD_NCH
            pltpu.sync_copy(idx_hbm.at[pl.ds(row0, PAD_NCH)], idx_v)
            bsel = s % 2

            if s >= 2:
                pltpu.make_async_copy(
                    buf.at[s % 2],
                    out_hbm.at[pl.ds((s - 2) * BF + base, B_PER_W)],
                    wsem).wait()

            @pl.loop(0, NCH)
            def _(j):
                pltpu.async_copy(table.at[idx_v.at[j]],
                                 buf.at[bsel, pl.ds(j * CHUNK, CHUNK)], sem)

            @pl.loop(0, NCH)
            def _(j):
                pltpu.make_async_copy(
                    table.at[idx_v.at[j]],
                    buf.at[bsel, pl.ds(j * CHUNK, CHUNK)], sem).wait()

            pltpu.async_copy(buf.at[bsel],
                             out_hbm.at[pl.ds(s * BF + base, B_PER_W)], wsem)

        for s in (NSTR - 2, NSTR - 1):
            pltpu.make_async_copy(
                buf.at[s % 2],
                out_hbm.at[pl.ds(s * BF + base, B_PER_W)], wsem).wait()

    return sc_kernel(idx_all, embt_flat, fw_flat)


def _mlp_body(fev_ref, yfw_ref, fv_ref, r26_ref, d1_ref, b1_ref, d2_ref,
              b2_ref, d3_ref, b3_ref, s_ref, w1_ref, w2_ref, w3_ref,
              bias_ref, out_ref):
    f32 = jnp.float32
    hi = lax.Precision.HIGHEST
    fv = fv_ref[...]
    fv_rep = lax.dot_general(fv, r26_ref[...], (((1,), (0,)), ((), ())),
                             precision=hi, preferred_element_type=f32)
    fev = fev_ref[...] * fv_rep

    acc = lax.dot_general(yfw_ref[...] * fv, w1_ref[...],
                          (((1,), (0,)), ((), ())),
                          precision=hi, preferred_element_type=f32)

    summed = lax.dot_general(fev, s_ref[...], (((1,), (0,)), ((), ())),
                             precision=hi, preferred_element_type=f32)
    part2 = lax.dot_general(fev * fev, s_ref[...], (((1,), (0,)), ((), ())),
                            precision=hi, preferred_element_type=f32)
    y2 = 0.5 * (summed * summed - part2)
    acc += lax.dot_general(y2, w2_ref[...], (((1,), (0,)), ((), ())),
                           precision=hi, preferred_element_type=f32)

    h = lax.dot_general(fev, d1_ref[...], (((1,), (0,)), ((), ())),
                        precision=hi, preferred_element_type=f32)
    h = jnp.maximum(h + b1_ref[...], 0.0)
    h = lax.dot_general(h, d2_ref[...], (((1,), (0,)), ((), ())),
                        precision=hi, preferred_element_type=f32)
    h = jnp.maximum(h + b2_ref[...], 0.0)
    h = lax.dot_general(h, d3_ref[...], (((1,), (0,)), ((), ())),
                        precision=hi, preferred_element_type=f32)
    h = jnp.maximum(h + b3_ref[...], 0.0)
    acc += lax.dot_general(h, w3_ref[...], (((1,), (0,)), ((), ())),
                           precision=hi, preferred_element_type=f32)
    out_ref[...] = acc + bias_ref[...]


def kernel(feat_index, feat_value, first_w, emb, d1_k, d1_b, bn1_g, bn1_b,
           d2_k, d2_b, bn2_g, bn2_b, d3_k, d3_b, bn3_g, bn3_b, fc_k, fc_b):
    f32 = jnp.float32
    i32 = jnp.int32
    idx = feat_index.reshape(NW, NCH, CHUNK).astype(i32)
    idx = jnp.pad(idx, ((0, 0), (0, PAD_NCH - NCH), (0, 0)))
    offs = jnp.concatenate([jnp.arange(E, dtype=i32) * V,
                            jnp.zeros((1,), i32)])
    idx_all = (idx[None, :, :, :] + offs[:, None, None, None]).reshape(
        NSTR * NW * PAD_NCH, CHUNK)

    embt_flat = jnp.concatenate([emb[:, e] for e in range(E)])
    fw_flat = first_w.reshape(-1)

    gath = _sc_gather(idx_all, embt_flat, fw_flat).reshape(NSTR, BF)

    fev_raw = gath[0:E].T.reshape(B, FE)
    yfw_raw = gath[E].reshape(B, F)
    fv = feat_value

    r26 = (lax.broadcasted_iota(i32, (F, FE), 0) ==
           lax.broadcasted_iota(i32, (F, FE), 1) // E).astype(f32)

    inv = 1.0 / jnp.sqrt(1.0 + EPS)
    d1 = d1_k * (bn1_g * inv)[None, :]
    b1 = (d1_b * bn1_g * inv + bn1_b)[None, :]
    d2 = d2_k * (bn2_g * inv)[None, :]
    b2 = (d2_b * bn2_g * inv + bn2_b)[None, :]
    d3 = d3_k * (bn3_g * inv)[None, :]
    b3 = (d3_b * bn3_g * inv + bn3_b)[None, :]

    w1 = fc_k[0:F, :]
    w2 = fc_k[F:F + E, :]
    w3 = fc_k[F + E:, :]
    bias = fc_b[None, :]

    s = (lax.broadcasted_iota(i32, (FE, E), 0) % E ==
         lax.broadcasted_iota(i32, (FE, E), 1)).astype(f32)

    grid = (B // BLK,)
    bspec = lambda w: pl.BlockSpec((BLK, w), lambda i: (i, 0))
    wspec = lambda shp: pl.BlockSpec(shp, lambda i: (0, 0))

    out = pl.pallas_call(
        _mlp_body,
        grid=grid,
        in_specs=[
            bspec(FE), bspec(F), bspec(F),
            wspec((F, FE)),
            wspec((FE, H)), wspec((1, H)),
            wspec((H, H)), wspec((1, H)),
            wspec((H, H)), wspec((1, H)),
            wspec((FE, E)),
            wspec((F, 1)), wspec((E, 1)), wspec((H, 1)), wspec((1, 1)),
        ],
        out_specs=pl.BlockSpec((BLK, 1), lambda i: (i, 0)),
        out_shape=jax.ShapeDtypeStruct((B, 1), f32),
    )(fev_raw, yfw_raw, fv, r26, d1, b1, d2, b2, d3, b3, s, w1, w2, w3, bias)
    return out


# speedup vs baseline: 4.1340x; 2.6180x over previous
import functools

import jax
import jax.numpy as jnp
from jax import lax
from jax.experimental import pallas as pl
from jax.experimental.pallas import tpu as pltpu
from jax.experimental.pallas import tpu_sc as plsc

B, F, V, E = 4096, 26, 1000000, 16
BF = B * F
FE = F * E
H = 400
EPS = 1e-3

NC, NS = 2, 16
NW = NC * NS
B_PER_W = BF // NW
CHUNK = 128
NCH = B_PER_W // CHUNK
PAD_NCH = 32
NSTR = E + 1
FCH = 4096
VP = 999936
TAILV = V - VP
NFCH = VP // FCH
FLAST = VP - NFCH * FCH
BLK = 512
assert B_PER_W % CHUNK == 0 and B % BLK == 0 and FLAST % 128 == 0


def _sc_flatten(embT, tail16):
    mesh = plsc.VectorSubcoreMesh(core_axis_name="c", subcore_axis_name="s")

    @functools.partial(
        pl.kernel,
        out_type=jax.ShapeDtypeStruct((E * V,), jnp.float32),
        mesh=mesh,
        scratch_types=[
            pltpu.VMEM((E, FCH), jnp.float32),
            pltpu.VMEM((2, FCH), jnp.float32),
            pltpu.SemaphoreType.DMA,
        ],
    )
    def flat_kernel(embt_hbm, tail_hbm, out_hbm, buf, stage, sem):
        wid = lax.axis_index("s") * NC + lax.axis_index("c")

        def do_chunk(col0, width):
            pltpu.sync_copy(embt_hbm.at[:, pl.ds(col0, width)],
                            buf.at[:, pl.ds(0, width)])
            for e in range(E):
                if e >= 2:
                    pltpu.make_async_copy(
                        stage.at[e % 2, pl.ds(0, width)],
                        out_hbm.at[pl.ds((e - 2) * VP + col0, width)],
                        sem).wait()

                @pl.loop(0, width // 128)
                def _(c2):
                    for u in range(8):
                        sl = pl.ds(c2 * 128 + u * 16, 16)
                        stage[e % 2, sl] = buf[e, sl]

                pltpu.async_copy(stage.at[e % 2, pl.ds(0, width)],
                                 out_hbm.at[pl.ds(e * VP + col0, width)],
                                 sem)
            for e in (E - 2, E - 1):
                pltpu.make_async_copy(
                    stage.at[e % 2, pl.ds(0, width)],
                    out_hbm.at[pl.ds(e * VP + col0, width)], sem).wait()

        @pl.loop(0, (NFCH + NW) // NW)
        def _(k):
            c = k * NW + wid

            @pl.when(c < NFCH)
            def _():
                do_chunk(c * FCH, FCH)

            @pl.when(c == NFCH)
            def _():
                do_chunk(NFCH * FCH, FLAST)

        @pl.when(wid == 1)
        def _():
            pltpu.sync_copy(tail_hbm, out_hbm.at[pl.ds(E * VP, E * TAILV)])

    return flat_kernel(embT, tail16)

    return flat_kernel(embT)


def _sc_gather(idx_all, embt_flat, fw_flat):
    mesh = plsc.VectorSubcoreMesh(core_axis_name="c", subcore_axis_name="s")

    @functools.partial(
        pl.kernel,
        out_type=jax.ShapeDtypeStruct((NSTR * BF,), jnp.float32),
        mesh=mesh,
        scratch_types=[
            pltpu.VMEM((PAD_NCH, CHUNK), jnp.int32),
            pltpu.VMEM((2, B_PER_W), jnp.float32),
            pltpu.SemaphoreType.DMA,
            pltpu.SemaphoreType.DMA,
        ],
    )
    def sc_kernel(idx_hbm, embt_hbm, fw_hbm, out_hbm, idx_v, buf, sem, wsem):
        wid = lax.axis_index("s") * NC + lax.axis_index("c")
        base = wid * B_PER_W

        for s in range(NSTR):
            table = embt_hbm if s < E else fw_hbm
            row0 = (s * NW + wid) * PAD_NCH
            pltpu.sync_copy(idx_hbm.at[pl.ds(row0, PAD_NCH)], idx_v)
            bsel = s % 2

            if s >= 2:
                pltpu.make_async_copy(
                    buf.at[s % 2],
                    out_hbm.at[pl.ds((s - 2) * BF + base, B_PER_W)],
                    wsem).wait()

            @pl.loop(0, NCH)
            def _(j):
                pltpu.async_copy(table.at[idx_v.at[j]],
                                 buf.at[bsel, pl.ds(j * CHUNK, CHUNK)], sem)

            @pl.loop(0, NCH)
            def _(j):
                pltpu.make_async_copy(
                    table.at[idx_v.at[j]],
                    buf.at[bsel, pl.ds(j * CHUNK, CHUNK)], sem).wait()

            pltpu.async_copy(buf.at[bsel],
                             out_hbm.at[pl.ds(s * BF + base, B_PER_W)], wsem)

        for s in (NSTR - 2, NSTR - 1):
            pltpu.make_async_copy(
                buf.at[s % 2],
                out_hbm.at[pl.ds(s * BF + base, B_PER_W)], wsem).wait()

    return sc_kernel(idx_all, embt_flat, fw_flat)


def _mlp_body(fev_ref, yfw_ref, fv_ref, r26_ref, d1_ref, b1_ref, d2_ref,
              b2_ref, d3_ref, b3_ref, s_ref, w1_ref, w2_ref, w3_ref,
              bias_ref, out_ref):
    f32 = jnp.float32
    hi = lax.Precision.HIGHEST
    fv = fv_ref[...]
    fv_rep = lax.dot_general(fv, r26_ref[...], (((1,), (0,)), ((), ())),
                             precision=hi, preferred_element_type=f32)
    fev = fev_ref[...] * fv_rep

    acc = lax.dot_general(yfw_ref[...] * fv, w1_ref[...],
                          (((1,), (0,)), ((), ())),
                          precision=hi, preferred_element_type=f32)

    summed = lax.dot_general(fev, s_ref[...], (((1,), (0,)), ((), ())),
                             precision=hi, preferred_element_type=f32)
    part2 = lax.dot_general(fev * fev, s_ref[...], (((1,), (0,)), ((), ())),
                            precision=hi, preferred_element_type=f32)
    y2 = 0.5 * (summed * summed - part2)
    acc += lax.dot_general(y2, w2_ref[...], (((1,), (0,)), ((), ())),
                           precision=hi, preferred_element_type=f32)

    h = lax.dot_general(fev, d1_ref[...], (((1,), (0,)), ((), ())),
                        precision=hi, preferred_element_type=f32)
    h = jnp.maximum(h + b1_ref[...], 0.0)
    h = lax.dot_general(h, d2_ref[...], (((1,), (0,)), ((), ())),
                        precision=hi, preferred_element_type=f32)
    h = jnp.maximum(h + b2_ref[...], 0.0)
    h = lax.dot_general(h, d3_ref[...], (((1,), (0,)), ((), ())),
                        precision=hi, preferred_element_type=f32)
    h = jnp.maximum(h + b3_ref[...], 0.0)
    acc += lax.dot_general(h, w3_ref[...], (((1,), (0,)), ((), ())),
                           precision=hi, preferred_element_type=f32)
    out_ref[...] = acc + bias_ref[...]


def kernel(feat_index, feat_value, first_w, emb, d1_k, d1_b, bn1_g, bn1_b,
           d2_k, d2_b, bn2_g, bn2_b, d3_k, d3_b, bn3_g, bn3_b, fc_k, fc_b):
    f32 = jnp.float32
    i32 = jnp.int32
    idx = feat_index.reshape(NW, NCH, CHUNK).astype(i32)
    idx = jnp.pad(idx, ((0, 0), (0, PAD_NCH - NCH), (0, 0)))
    e_ax = jnp.arange(E, dtype=i32)[:, None, None, None]
    idx4 = idx[None, :, :, :]
    emb_pos = jnp.where(idx4 < VP, e_ax * VP + idx4,
                        E * VP + e_ax * TAILV + (idx4 - VP))
    idx_all = jnp.concatenate([emb_pos, idx4], axis=0).reshape(
        NSTR * NW * PAD_NCH, CHUNK)

    tail16 = emb[VP:, :].T.reshape(-1)
    embt_flat = _sc_flatten(emb.T, tail16)
    fw_flat = first_w.reshape(-1)

    gath = _sc_gather(idx_all, embt_flat, fw_flat).reshape(NSTR, BF)

    fev_raw = gath[0:E].T.reshape(B, FE)
    yfw_raw = gath[E].reshape(B, F)
    fv = feat_value

    r26 = (lax.broadcasted_iota(i32, (F, FE), 0) ==
           lax.broadcasted_iota(i32, (F, FE), 1) // E).astype(f32)

    inv = 1.0 / jnp.sqrt(1.0 + EPS)
    d1 = d1_k * (bn1_g * inv)[None, :]
    b1 = (d1_b * bn1_g * inv + bn1_b)[None, :]
    d2 = d2_k * (bn2_g * inv)[None, :]
    b2 = (d2_b * bn2_g * inv + bn2_b)[None, :]
    d3 = d3_k * (bn3_g * inv)[None, :]
    b3 = (d3_b * bn3_g * inv + bn3_b)[None, :]

    w1 = fc_k[0:F, :]
    w2 = fc_k[F:F + E, :]
    w3 = fc_k[F + E:, :]
    bias = fc_b[None, :]

    s = (lax.broadcasted_iota(i32, (FE, E), 0) % E ==
         lax.broadcasted_iota(i32, (FE, E), 1)).astype(f32)

    grid = (B // BLK,)
    bspec = lambda w: pl.BlockSpec((BLK, w), lambda i: (i, 0))
    wspec = lambda shp: pl.BlockSpec(shp, lambda i: (0, 0))

    out = pl.pallas_call(
        _mlp_body,
        grid=grid,
        in_specs=[
            bspec(FE), bspec(F), bspec(F),
            wspec((F, FE)),
            wspec((FE, H)), wspec((1, H)),
            wspec((H, H)), wspec((1, H)),
            wspec((H, H)), wspec((1, H)),
            wspec((FE, E)),
            wspec((F, 1)), wspec((E, 1)), wspec((H, 1)), wspec((1, 1)),
        ],
        out_specs=pl.BlockSpec((BLK, 1), lambda i: (i, 0)),
        out_shape=jax.ShapeDtypeStruct((B, 1), f32),
    )(fev_raw, yfw_raw, fv, r26, d1, b1, d2, b2, d3, b3, s, w1, w2, w3, bias)
    return out


# speedup vs baseline: 5.0793x; 1.2286x over previous
import functools

import jax
import jax.numpy as jnp
from jax import lax
from jax.experimental import pallas as pl
from jax.experimental.pallas import tpu as pltpu
from jax.experimental.pallas import tpu_sc as plsc

B, F, V, E = 4096, 26, 1000000, 16
BF = B * F
FE = F * E
H = 400
EPS = 1e-3

NC, NS = 2, 16
NW = NC * NS
B_PER_W = BF // NW
CHUNK = 128
NCH = B_PER_W // CHUNK
PAD_NCH = 32
NSTR = E + 1
FCH = 512
VP = 999936
TAILV = V - VP
NFCH = VP // FCH
NCPT = (NFCH + NW - 1) // NW + 1
BLK = 512
assert B_PER_W % CHUNK == 0 and B % BLK == 0 and VP % FCH == 0


def _sc_flatten(embT, tail16):
    mesh = plsc.VectorSubcoreMesh(core_axis_name="c", subcore_axis_name="s")

    @functools.partial(
        pl.kernel,
        out_type=jax.ShapeDtypeStruct((E * V,), jnp.float32),
        mesh=mesh,
        scratch_types=[
            pltpu.VMEM((2, E, FCH), jnp.float32),
            pltpu.VMEM((2, FCH), jnp.float32),
            pltpu.SemaphoreType.DMA,
            pltpu.SemaphoreType.DMA,
        ],
    )
    def flat_kernel(embt_hbm, tail_hbm, out_hbm, buf, stage, sem, insem):
        wid = lax.axis_index("s") * NC + lax.axis_index("c")

        @pl.when(wid < NFCH)
        def _():
            pltpu.async_copy(embt_hbm.at[:, pl.ds(wid * FCH, FCH)],
                             buf.at[0], insem)

        @pl.loop(0, NCPT)
        def _(k):
            c = k * NW + wid

            @pl.when(c < NFCH)
            def _():
                cn = c + NW

                @pl.when(cn < NFCH)
                def _():
                    pltpu.async_copy(embt_hbm.at[:, pl.ds(cn * FCH, FCH)],
                                     buf.at[(k + 1) % 2], insem)

                pltpu.make_async_copy(embt_hbm.at[:, pl.ds(c * FCH, FCH)],
                                      buf.at[k % 2], insem).wait()
                col0 = c * FCH
                for e in range(E):
                    if e >= 2:
                        pltpu.make_async_copy(
                            stage.at[e % 2],
                            out_hbm.at[pl.ds((e - 2) * VP + col0, FCH)],
                            sem).wait()

                    @pl.loop(0, FCH // 128)
                    def _(c2):
                        for u in range(8):
                            sl = pl.ds(c2 * 128 + u * 16, 16)
                            stage[e % 2, sl] = buf[k % 2, e, sl]

                    pltpu.async_copy(stage.at[e % 2],
                                     out_hbm.at[pl.ds(e * VP + col0, FCH)],
                                     sem)
                for e in (E - 2, E - 1):
                    pltpu.make_async_copy(
                        stage.at[e % 2],
                        out_hbm.at[pl.ds(e * VP + col0, FCH)], sem).wait()

        @pl.when(wid == 1)
        def _():
            pltpu.sync_copy(tail_hbm, out_hbm.at[pl.ds(E * VP, E * TAILV)])

    return flat_kernel(embT, tail16)

    return flat_kernel(embT)


def _sc_gather(idx_all, embt_flat, fw_flat):
    mesh = plsc.VectorSubcoreMesh(core_axis_name="c", subcore_axis_name="s")

    @functools.partial(
        pl.kernel,
        out_type=jax.ShapeDtypeStruct((NSTR * BF,), jnp.float32),
        mesh=mesh,
        scratch_types=[
            pltpu.VMEM((PAD_NCH, CHUNK), jnp.int32),
            pltpu.VMEM((2, B_PER_W), jnp.float32),
            pltpu.SemaphoreType.DMA,
            pltpu.SemaphoreType.DMA,
        ],
    )
    def sc_kernel(idx_hbm, embt_hbm, fw_hbm, out_hbm, idx_v, buf, sem, wsem):
        wid = lax.axis_index("s") * NC + lax.axis_index("c")
        base = wid * B_PER_W

        for s in range(NSTR):
            table = embt_hbm if s < E else fw_hbm
            row0 = (s * NW + wid) * PAD_NCH
            pltpu.sync_copy(idx_hbm.at[pl.ds(row0, PAD_NCH)], idx_v)
            bsel = s % 2

            if s >= 2:
                pltpu.make_async_copy(
                    buf.at[s % 2],
                    out_hbm.at[pl.ds((s - 2) * BF + base, B_PER_W)],
                    wsem).wait()

            @pl.loop(0, NCH)
            def _(j):
                pltpu.async_copy(table.at[idx_v.at[j]],
                                 buf.at[bsel, pl.ds(j * CHUNK, CHUNK)], sem)

            @pl.loop(0, NCH)
            def _(j):
                pltpu.make_async_copy(
                    table.at[idx_v.at[j]],
                    buf.at[bsel, pl.ds(j * CHUNK, CHUNK)], sem).wait()

            pltpu.async_copy(buf.at[bsel],
                             out_hbm.at[pl.ds(s * BF + base, B_PER_W)], wsem)

        for s in (NSTR - 2, NSTR - 1):
            pltpu.make_async_copy(
                buf.at[s % 2],
                out_hbm.at[pl.ds(s * BF + base, B_PER_W)], wsem).wait()

    return sc_kernel(idx_all, embt_flat, fw_flat)


def _mlp_body(fev_ref, yfw_ref, fv_ref, r26_ref, d1_ref, b1_ref, d2_ref,
              b2_ref, d3_ref, b3_ref, s_ref, w1_ref, w2_ref, w3_ref,
              bias_ref, out_ref):
    f32 = jnp.float32
    hi = lax.Precision.HIGHEST
    fv = fv_ref[...]
    fv_rep = lax.dot_general(fv, r26_ref[...], (((1,), (0,)), ((), ())),
                             precision=hi, preferred_element_type=f32)
    fev = fev_ref[...] * fv_rep

    acc = lax.dot_general(yfw_ref[...] * fv, w1_ref[...],
                          (((1,), (0,)), ((), ())),
                          precision=hi, preferred_element_type=f32)

    bf16 = jnp.bfloat16
    fev_b = fev.astype(bf16)
    s_b = s_ref[...].astype(bf16)
    summed = lax.dot_general(fev_b, s_b, (((1,), (0,)), ((), ())),
                             preferred_element_type=f32)
    part2 = lax.dot_general((fev * fev).astype(bf16), s_b,
                            (((1,), (0,)), ((), ())),
                            preferred_element_type=f32)
    y2 = 0.5 * (summed * summed - part2)
    acc += lax.dot_general(y2, w2_ref[...], (((1,), (0,)), ((), ())),
                           precision=hi, preferred_element_type=f32)

    h = lax.dot_general(fev_b, d1_ref[...].astype(bf16),
                        (((1,), (0,)), ((), ())),
                        preferred_element_type=f32)
    h = jnp.maximum(h + b1_ref[...], 0.0)
    h = lax.dot_general(h.astype(bf16), d2_ref[...].astype(bf16),
                        (((1,), (0,)), ((), ())),
                        preferred_element_type=f32)
    h = jnp.maximum(h + b2_ref[...], 0.0)
    h = lax.dot_general(h.astype(bf16), d3_ref[...].astype(bf16),
                        (((1,), (0,)), ((), ())),
                        preferred_element_type=f32)
    h = jnp.maximum(h + b3_ref[...], 0.0)
    acc += lax.dot_general(h, w3_ref[...], (((1,), (0,)), ((), ())),
                           precision=hi, preferred_element_type=f32)
    out_ref[...] = acc + bias_ref[...]


def kernel(feat_index, feat_value, first_w, emb, d1_k, d1_b, bn1_g, bn1_b,
           d2_k, d2_b, bn2_g, bn2_b, d3_k, d3_b, bn3_g, bn3_b, fc_k, fc_b):
    f32 = jnp.float32
    i32 = jnp.int32
    idx = feat_index.reshape(NW, NCH, CHUNK).astype(i32)
    idx = jnp.pad(idx, ((0, 0), (0, PAD_NCH - NCH), (0, 0)))
    e_ax = jnp.arange(E, dtype=i32)[:, None, None, None]
    idx4 = idx[None, :, :, :]
    emb_pos = jnp.where(idx4 < VP, e_ax * VP + idx4,
                        E * VP + e_ax * TAILV + (idx4 - VP))
    idx_all = jnp.concatenate([emb_pos, idx4], axis=0).reshape(
        NSTR * NW * PAD_NCH, CHUNK)

    tail16 = emb[VP:, :].T.reshape(-1)
    embt_flat = _sc_flatten(emb.T, tail16)
    fw_flat = first_w.reshape(-1)

    gath = _sc_gather(idx_all, embt_flat, fw_flat).reshape(NSTR, BF)

    fev_raw = gath[0:E].T.reshape(B, FE)
    yfw_raw = gath[E].reshape(B, F)
    fv = feat_value

    r26 = (lax.broadcasted_iota(i32, (F, FE), 0) ==
           lax.broadcasted_iota(i32, (F, FE), 1) // E).astype(f32)

    inv = 1.0 / jnp.sqrt(1.0 + EPS)
    d1 = d1_k * (bn1_g * inv)[None, :]
    b1 = (d1_b * bn1_g * inv + bn1_b)[None, :]
    d2 = d2_k * (bn2_g * inv)[None, :]
    b2 = (d2_b * bn2_g * inv + bn2_b)[None, :]
    d3 = d3_k * (bn3_g * inv)[None, :]
    b3 = (d3_b * bn3_g * inv + bn3_b)[None, :]

    w1 = fc_k[0:F, :]
    w2 = fc_k[F:F + E, :]
    w3 = fc_k[F + E:, :]
    bias = fc_b[None, :]

    s = (lax.broadcasted_iota(i32, (FE, E), 0) % E ==
         lax.broadcasted_iota(i32, (FE, E), 1)).astype(f32)

    grid = (B // BLK,)
    bspec = lambda w: pl.BlockSpec((BLK, w), lambda i: (i, 0))
    wspec = lambda shp: pl.BlockSpec(shp, lambda i: (0, 0))

    out = pl.pallas_call(
        _mlp_body,
        grid=grid,
        in_specs=[
            bspec(FE), bspec(F), bspec(F),
            wspec((F, FE)),
            wspec((FE, H)), wspec((1, H)),
            wspec((H, H)), wspec((1, H)),
            wspec((H, H)), wspec((1, H)),
            wspec((FE, E)),
            wspec((F, 1)), wspec((E, 1)), wspec((H, 1)), wspec((1, 1)),
        ],
        out_specs=pl.BlockSpec((BLK, 1), lambda i: (i, 0)),
        out_shape=jax.ShapeDtypeStruct((B, 1), f32),
    )(fev_raw, yfw_raw, fv, r26, d1, b1, d2, b2, d3, b3, s, w1, w2, w3, bias)
    return out


# speedup vs baseline: 5.3386x; 1.0511x over previous
import functools

import jax
import jax.numpy as jnp
from jax import lax
from jax.experimental import pallas as pl
from jax.experimental.pallas import tpu as pltpu
from jax.experimental.pallas import tpu_sc as plsc

B, F, V, E = 4096, 26, 1000000, 16
BF = B * F
FE = F * E
H = 400
EPS = 1e-3

NC, NS = 2, 16
NW = NC * NS
B_PER_W = BF // NW
CHUNK = 128
NCH = B_PER_W // CHUNK
PAD_NCH = 32
NSTR = E + 1
FCH = 512
VP = 999936
TAILV = V - VP
NFCH = VP // FCH
NCPT = (NFCH + NW - 1) // NW + 1
BLK = 512
assert B_PER_W % CHUNK == 0 and B % BLK == 0 and VP % FCH == 0


def _sc_flatten(embT, tail16):
    mesh = plsc.VectorSubcoreMesh(core_axis_name="c", subcore_axis_name="s")

    @functools.partial(
        pl.kernel,
        out_type=jax.ShapeDtypeStruct((E * V,), jnp.float32),
        mesh=mesh,
        scratch_types=[
            pltpu.VMEM((2, E, FCH), jnp.float32),
            pltpu.VMEM((2, FCH), jnp.float32),
            pltpu.SemaphoreType.DMA,
            pltpu.SemaphoreType.DMA,
        ],
    )
    def flat_kernel(embt_hbm, tail_hbm, out_hbm, buf, stage, sem, insem):
        wid = lax.axis_index("s") * NC + lax.axis_index("c")

        @pl.when(wid < NFCH)
        def _():
            pltpu.async_copy(embt_hbm.at[:, pl.ds(wid * FCH, FCH)],
                             buf.at[0], insem)

        @pl.loop(0, NCPT)
        def _(k):
            c = k * NW + wid

            @pl.when(c < NFCH)
            def _():
                cn = c + NW

                @pl.when(cn < NFCH)
                def _():
                    pltpu.async_copy(embt_hbm.at[:, pl.ds(cn * FCH, FCH)],
                                     buf.at[(k + 1) % 2], insem)

                pltpu.make_async_copy(embt_hbm.at[:, pl.ds(c * FCH, FCH)],
                                      buf.at[k % 2], insem).wait()
                col0 = c * FCH
                for e in range(E):
                    if e >= 2:
                        pltpu.make_async_copy(
                            stage.at[e % 2],
                            out_hbm.at[pl.ds((e - 2) * VP + col0, FCH)],
                            sem).wait()

                    @pl.loop(0, FCH // 128)
                    def _(c2):
                        for u in range(8):
                            sl = pl.ds(c2 * 128 + u * 16, 16)
                            stage[e % 2, sl] = buf[k % 2, e, sl]

                    pltpu.async_copy(stage.at[e % 2],
                                     out_hbm.at[pl.ds(e * VP + col0, FCH)],
                                     sem)
                for e in (E - 2, E - 1):
                    pltpu.make_async_copy(
                        stage.at[e % 2],
                        out_hbm.at[pl.ds(e * VP + col0, FCH)], sem).wait()

        @pl.when(wid == 1)
        def _():
            pltpu.sync_copy(tail_hbm, out_hbm.at[pl.ds(E * VP, E * TAILV)])

    return flat_kernel(embT, tail16)

    return flat_kernel(embT)


def _sc_gather(idx_all, embt_flat, fw_flat):
    mesh = plsc.VectorSubcoreMesh(core_axis_name="c", subcore_axis_name="s")

    @functools.partial(
        pl.kernel,
        out_type=jax.ShapeDtypeStruct((NSTR * BF,), jnp.float32),
        mesh=mesh,
        scratch_types=[
            pltpu.VMEM((PAD_NCH, CHUNK), jnp.int32),
            pltpu.VMEM((PAD_NCH, CHUNK), jnp.int32),
            pltpu.VMEM((PAD_NCH, CHUNK), jnp.int32),
            pltpu.VMEM((B_PER_W,), jnp.float32),
            pltpu.VMEM((B_PER_W,), jnp.float32),
            pltpu.VMEM((B_PER_W,), jnp.float32),
            pltpu.SemaphoreType.DMA,
            pltpu.SemaphoreType.DMA,
            pltpu.SemaphoreType.DMA,
        ],
    )
    def sc_kernel(idx_hbm, embt_hbm, fw_hbm, out_hbm, iv0, iv1, iv2,
                  b0, b1, b2, sem, wsem, isem):
        wid = lax.axis_index("s") * NC + lax.axis_index("c")
        base = wid * B_PER_W
        ivs = (iv0, iv1, iv2)
        bufs = (b0, b1, b2)

        def idx_rows(s):
            return idx_hbm.at[pl.ds((s * NW + wid) * PAD_NCH, PAD_NCH)]

        def fire(s):
            table = embt_hbm if s < E else fw_hbm
            iv, bf = ivs[s % 3], bufs[s % 3]

            @pl.loop(0, NCH)
            def _(j):
                pltpu.async_copy(table.at[iv.at[j]],
                                 bf.at[pl.ds(j * CHUNK, CHUNK)], sem)

        def drain(s):
            table = embt_hbm if s < E else fw_hbm
            iv, bf = ivs[s % 3], bufs[s % 3]

            @pl.loop(0, NCH)
            def _(j):
                pltpu.make_async_copy(table.at[iv.at[j]],
                                      bf.at[pl.ds(j * CHUNK, CHUNK)],
                                      sem).wait()

        pltpu.sync_copy(idx_rows(0), iv0)
        pltpu.async_copy(idx_rows(1), iv1, isem)
        fire(0)

        for s in range(NSTR):
            if s + 1 < NSTR:
                if s + 2 < NSTR:
                    pltpu.async_copy(idx_rows(s + 2), ivs[(s + 2) % 3], isem)
                pltpu.make_async_copy(idx_rows(s + 1), ivs[(s + 1) % 3],
                                      isem).wait()
                if s >= 2:
                    pltpu.make_async_copy(
                        bufs[(s + 1) % 3],
                        out_hbm.at[pl.ds((s - 2) * BF + base, B_PER_W)],
                        wsem).wait()
                fire(s + 1)
            drain(s)
            pltpu.async_copy(bufs[s % 3],
                             out_hbm.at[pl.ds(s * BF + base, B_PER_W)], wsem)

        for s in (NSTR - 3, NSTR - 2, NSTR - 1):
            pltpu.make_async_copy(
                bufs[s % 3],
                out_hbm.at[pl.ds(s * BF + base, B_PER_W)], wsem).wait()

    return sc_kernel(idx_all, embt_flat, fw_flat)


def _mlp_body(fev_ref, yfw_ref, fv_ref, r26_ref, d1_ref, b1_ref, d2_ref,
              b2_ref, d3_ref, b3_ref, s_ref, w1_ref, w2_ref, w3_ref,
              bias_ref, out_ref):
    f32 = jnp.float32
    hi = lax.Precision.HIGHEST
    fv = fv_ref[...]
    fv_rep = lax.dot_general(fv, r26_ref[...], (((1,), (0,)), ((), ())),
                             precision=hi, preferred_element_type=f32)
    fev = fev_ref[...] * fv_rep

    acc = lax.dot_general(yfw_ref[...] * fv, w1_ref[...],
                          (((1,), (0,)), ((), ())),
                          precision=hi, preferred_element_type=f32)

    bf16 = jnp.bfloat16
    fev_b = fev.astype(bf16)
    s_b = s_ref[...].astype(bf16)
    summed = lax.dot_general(fev_b, s_b, (((1,), (0,)), ((), ())),
                             preferred_element_type=f32)
    part2 = lax.dot_general((fev * fev).astype(bf16), s_b,
                            (((1,), (0,)), ((), ())),
                            preferred_element_type=f32)
    y2 = 0.5 * (summed * summed - part2)
    acc += lax.dot_general(y2, w2_ref[...], (((1,), (0,)), ((), ())),
                           precision=hi, preferred_element_type=f32)

    h = lax.dot_general(fev_b, d1_ref[...].astype(bf16),
                        (((1,), (0,)), ((), ())),
                        preferred_element_type=f32)
    h = jnp.maximum(h + b1_ref[...], 0.0)
    h = lax.dot_general(h.astype(bf16), d2_ref[...].astype(bf16),
                        (((1,), (0,)), ((), ())),
                        preferred_element_type=f32)
    h = jnp.maximum(h + b2_ref[...], 0.0)
    h = lax.dot_general(h.astype(bf16), d3_ref[...].astype(bf16),
                        (((1,), (0,)), ((), ())),
                        preferred_element_type=f32)
    h = jnp.maximum(h + b3_ref[...], 0.0)
    acc += lax.dot_general(h, w3_ref[...], (((1,), (0,)), ((), ())),
                           precision=hi, preferred_element_type=f32)
    out_ref[...] = acc + bias_ref[...]


def kernel(feat_index, feat_value, first_w, emb, d1_k, d1_b, bn1_g, bn1_b,
           d2_k, d2_b, bn2_g, bn2_b, d3_k, d3_b, bn3_g, bn3_b, fc_k, fc_b):
    f32 = jnp.float32
    i32 = jnp.int32
    idx = feat_index.reshape(NW, NCH, CHUNK).astype(i32)
    idx = jnp.pad(idx, ((0, 0), (0, PAD_NCH - NCH), (0, 0)))
    e_ax = jnp.arange(E, dtype=i32)[:, None, None, None]
    idx4 = idx[None, :, :, :]
    emb_pos = jnp.where(idx4 < VP, e_ax * VP + idx4,
                        E * VP + e_ax * TAILV + (idx4 - VP))
    idx_all = jnp.concatenate([emb_pos, idx4], axis=0).reshape(
        NSTR * NW * PAD_NCH, CHUNK)

    tail16 = emb[VP:, :].T.reshape(-1)
    embt_flat = _sc_flatten(emb.T, tail16)
    fw_flat = first_w.reshape(-1)

    gath = _sc_gather(idx_all, embt_flat, fw_flat).reshape(NSTR, BF)

    fev_raw = gath[0:E].T.reshape(B, FE)
    yfw_raw = gath[E].reshape(B, F)
    fv = feat_value

    r26 = (lax.broadcasted_iota(i32, (F, FE), 0) ==
           lax.broadcasted_iota(i32, (F, FE), 1) // E).astype(f32)

    inv = 1.0 / jnp.sqrt(1.0 + EPS)
    d1 = d1_k * (bn1_g * inv)[None, :]
    b1 = (d1_b * bn1_g * inv + bn1_b)[None, :]
    d2 = d2_k * (bn2_g * inv)[None, :]
    b2 = (d2_b * bn2_g * inv + bn2_b)[None, :]
    d3 = d3_k * (bn3_g * inv)[None, :]
    b3 = (d3_b * bn3_g * inv + bn3_b)[None, :]

    w1 = fc_k[0:F, :]
    w2 = fc_k[F:F + E, :]
    w3 = fc_k[F + E:, :]
    bias = fc_b[None, :]

    s = (lax.broadcasted_iota(i32, (FE, E), 0) % E ==
         lax.broadcasted_iota(i32, (FE, E), 1)).astype(f32)

    grid = (B // BLK,)
    bspec = lambda w: pl.BlockSpec((BLK, w), lambda i: (i, 0))
    wspec = lambda shp: pl.BlockSpec(shp, lambda i: (0, 0))

    out = pl.pallas_call(
        _mlp_body,
        grid=grid,
        in_specs=[
            bspec(FE), bspec(F), bspec(F),
            wspec((F, FE)),
            wspec((FE, H)), wspec((1, H)),
            wspec((H, H)), wspec((1, H)),
            wspec((H, H)), wspec((1, H)),
            wspec((FE, E)),
            wspec((F, 1)), wspec((E, 1)), wspec((H, 1)), wspec((1, 1)),
        ],
        out_specs=pl.BlockSpec((BLK, 1), lambda i: (i, 0)),
        out_shape=jax.ShapeDtypeStruct((B, 1), f32),
    )(fev_raw, yfw_raw, fv, r26, d1, b1, d2, b2, d3, b3, s, w1, w2, w3, bias)
    return out


# speedup vs baseline: 5.3493x; 1.0020x over previous
import functools

import jax
import jax.numpy as jnp
from jax import lax
from jax.experimental import pallas as pl
from jax.experimental.pallas import tpu as pltpu
from jax.experimental.pallas import tpu_sc as plsc

B, F, V, E = 4096, 26, 1000000, 16
BF = B * F
FE = F * E
H = 400
EPS = 1e-3

NC, NS = 2, 16
NW = NC * NS
B_PER_W = BF // NW
CHUNK = 128
NCH = B_PER_W // CHUNK
PAD_NCH = 32
NSTR = E + 1
FCH = 512
VP = 999936
TAILV = V - VP
NFCH = VP // FCH
NCPT = (NFCH + NW - 1) // NW + 1
BLK = 512
assert B_PER_W % CHUNK == 0 and B % BLK == 0 and VP % FCH == 0


def _sc_flatten(embT, tail16):
    mesh = plsc.VectorSubcoreMesh(core_axis_name="c", subcore_axis_name="s")

    @functools.partial(
        pl.kernel,
        out_type=jax.ShapeDtypeStruct((E * V,), jnp.float32),
        mesh=mesh,
        scratch_types=[
            pltpu.VMEM((2, E, FCH), jnp.float32),
            pltpu.VMEM((2, FCH), jnp.float32),
            pltpu.SemaphoreType.DMA,
            pltpu.SemaphoreType.DMA,
        ],
    )
    def flat_kernel(embt_hbm, tail_hbm, out_hbm, buf, stage, sem, insem):
        wid = lax.axis_index("s") * NC + lax.axis_index("c")

        @pl.when(wid < NFCH)
        def _():
            pltpu.async_copy(embt_hbm.at[:, pl.ds(wid * FCH, FCH)],
                             buf.at[0], insem)

        @pl.loop(0, NCPT)
        def _(k):
            c = k * NW + wid

            @pl.when(c < NFCH)
            def _():
                cn = c + NW

                @pl.when(cn < NFCH)
                def _():
                    pltpu.async_copy(embt_hbm.at[:, pl.ds(cn * FCH, FCH)],
                                     buf.at[(k + 1) % 2], insem)

                pltpu.make_async_copy(embt_hbm.at[:, pl.ds(c * FCH, FCH)],
                                      buf.at[k % 2], insem).wait()
                col0 = c * FCH
                for e in range(E):
                    if e >= 2:
                        pltpu.make_async_copy(
                            stage.at[e % 2],
                            out_hbm.at[pl.ds((e - 2) * VP + col0, FCH)],
                            sem).wait()

                    for u in range(FCH // 16):
                        sl = pl.ds(u * 16, 16)
                        stage[e % 2, sl] = buf[k % 2, e, sl]

                    pltpu.async_copy(stage.at[e % 2],
                                     out_hbm.at[pl.ds(e * VP + col0, FCH)],
                                     sem)
                for e in (E - 2, E - 1):
                    pltpu.make_async_copy(
                        stage.at[e % 2],
                        out_hbm.at[pl.ds(e * VP + col0, FCH)], sem).wait()

        @pl.when(wid == 1)
        def _():
            pltpu.sync_copy(tail_hbm, out_hbm.at[pl.ds(E * VP, E * TAILV)])

    return flat_kernel(embT, tail16)

    return flat_kernel(embT)


def _sc_gather(idx_all, embt_flat, fw_flat):
    mesh = plsc.VectorSubcoreMesh(core_axis_name="c", subcore_axis_name="s")

    @functools.partial(
        pl.kernel,
        out_type=jax.ShapeDtypeStruct((NSTR * BF,), jnp.float32),
        mesh=mesh,
        scratch_types=[
            pltpu.VMEM((PAD_NCH, CHUNK), jnp.int32),
            pltpu.VMEM((PAD_NCH, CHUNK), jnp.int32),
            pltpu.VMEM((PAD_NCH, CHUNK), jnp.int32),
            pltpu.VMEM((B_PER_W,), jnp.float32),
            pltpu.VMEM((B_PER_W,), jnp.float32),
            pltpu.VMEM((B_PER_W,), jnp.float32),
            pltpu.SemaphoreType.DMA,
            pltpu.SemaphoreType.DMA,
            pltpu.SemaphoreType.DMA,
        ],
    )
    def sc_kernel(idx_hbm, embt_hbm, fw_hbm, out_hbm, iv0, iv1, iv2,
                  b0, b1, b2, sem, wsem, isem):
        wid = lax.axis_index("s") * NC + lax.axis_index("c")
        base = wid * B_PER_W
        ivs = (iv0, iv1, iv2)
        bufs = (b0, b1, b2)

        def idx_rows(s):
            return idx_hbm.at[pl.ds((s * NW + wid) * PAD_NCH, PAD_NCH)]

        def fire(s):
            table = embt_hbm if s < E else fw_hbm
            iv, bf = ivs[s % 3], bufs[s % 3]

            @pl.loop(0, NCH)
            def _(j):
                pltpu.async_copy(table.at[iv.at[j]],
                                 bf.at[pl.ds(j * CHUNK, CHUNK)], sem)

        def drain(s):
            table = embt_hbm if s < E else fw_hbm
            iv, bf = ivs[s % 3], bufs[s % 3]

            @pl.loop(0, NCH)
            def _(j):
                pltpu.make_async_copy(table.at[iv.at[j]],
                                      bf.at[pl.ds(j * CHUNK, CHUNK)],
                                      sem).wait()

        pltpu.sync_copy(idx_rows(0), iv0)
        pltpu.async_copy(idx_rows(1), iv1, isem)
        fire(0)

        for s in range(NSTR):
            if s + 1 < NSTR:
                if s + 2 < NSTR:
                    pltpu.async_copy(idx_rows(s + 2), ivs[(s + 2) % 3], isem)
                pltpu.make_async_copy(idx_rows(s + 1), ivs[(s + 1) % 3],
                                      isem).wait()
                if s >= 2:
                    pltpu.make_async_copy(
                        bufs[(s + 1) % 3],
                        out_hbm.at[pl.ds((s - 2) * BF + base, B_PER_W)],
                        wsem).wait()
                fire(s + 1)
            drain(s)
            pltpu.async_copy(bufs[s % 3],
                             out_hbm.at[pl.ds(s * BF + base, B_PER_W)], wsem)

        for s in (NSTR - 3, NSTR - 2, NSTR - 1):
            pltpu.make_async_copy(
                bufs[s % 3],
                out_hbm.at[pl.ds(s * BF + base, B_PER_W)], wsem).wait()

    return sc_kernel(idx_all, embt_flat, fw_flat)


def _mlp_body(fev_ref, yfw_ref, fv_ref, r26_ref, d1_ref, b1_ref, d2_ref,
              b2_ref, d3_ref, b3_ref, s_ref, w1_ref, w2_ref, w3_ref,
              bias_ref, out_ref):
    f32 = jnp.float32
    hi = lax.Precision.HIGHEST
    fv = fv_ref[...]
    fv_rep = lax.dot_general(fv, r26_ref[...], (((1,), (0,)), ((), ())),
                             precision=hi, preferred_element_type=f32)
    fev = fev_ref[...] * fv_rep

    acc = lax.dot_general(yfw_ref[...] * fv, w1_ref[...],
                          (((1,), (0,)), ((), ())),
                          precision=hi, preferred_element_type=f32)

    bf16 = jnp.bfloat16
    fev_b = fev.astype(bf16)
    s_b = s_ref[...].astype(bf16)
    summed = lax.dot_general(fev_b, s_b, (((1,), (0,)), ((), ())),
                             preferred_element_type=f32)
    part2 = lax.dot_general((fev * fev).astype(bf16), s_b,
                            (((1,), (0,)), ((), ())),
                            preferred_element_type=f32)
    y2 = 0.5 * (summed * summed - part2)
    acc += lax.dot_general(y2, w2_ref[...], (((1,), (0,)), ((), ())),
                           precision=hi, preferred_element_type=f32)

    h = lax.dot_general(fev_b, d1_ref[...].astype(bf16),
                        (((1,), (0,)), ((), ())),
                        preferred_element_type=f32)
    h = jnp.maximum(h + b1_ref[...], 0.0)
    h = lax.dot_general(h.astype(bf16), d2_ref[...].astype(bf16),
                        (((1,), (0,)), ((), ())),
                        preferred_element_type=f32)
    h = jnp.maximum(h + b2_ref[...], 0.0)
    h = lax.dot_general(h.astype(bf16), d3_ref[...].astype(bf16),
                        (((1,), (0,)), ((), ())),
                        preferred_element_type=f32)
    h = jnp.maximum(h + b3_ref[...], 0.0)
    acc += lax.dot_general(h, w3_ref[...], (((1,), (0,)), ((), ())),
                           precision=hi, preferred_element_type=f32)
    out_ref[...] = acc + bias_ref[...]


def kernel(feat_index, feat_value, first_w, emb, d1_k, d1_b, bn1_g, bn1_b,
           d2_k, d2_b, bn2_g, bn2_b, d3_k, d3_b, bn3_g, bn3_b, fc_k, fc_b):
    f32 = jnp.float32
    i32 = jnp.int32
    idx = feat_index.reshape(NW, NCH, CHUNK).astype(i32)
    idx = jnp.pad(idx, ((0, 0), (0, PAD_NCH - NCH), (0, 0)))
    e_ax = jnp.arange(E, dtype=i32)[:, None, None, None]
    idx4 = idx[None, :, :, :]
    emb_pos = jnp.where(idx4 < VP, e_ax * VP + idx4,
                        E * VP + e_ax * TAILV + (idx4 - VP))
    idx_all = jnp.concatenate([emb_pos, idx4], axis=0).reshape(
        NSTR * NW * PAD_NCH, CHUNK)

    tail16 = emb[VP:, :].T.reshape(-1)
    embt_flat = _sc_flatten(emb.T, tail16)
    fw_flat = first_w.reshape(-1)

    gath = _sc_gather(idx_all, embt_flat, fw_flat).reshape(NSTR, BF)

    fev_raw = gath[0:E].T.reshape(B, FE)
    yfw_raw = gath[E].reshape(B, F)
    fv = feat_value

    r26 = (lax.broadcasted_iota(i32, (F, FE), 0) ==
           lax.broadcasted_iota(i32, (F, FE), 1) // E).astype(f32)

    inv = 1.0 / jnp.sqrt(1.0 + EPS)
    d1 = d1_k * (bn1_g * inv)[None, :]
    b1 = (d1_b * bn1_g * inv + bn1_b)[None, :]
    d2 = d2_k * (bn2_g * inv)[None, :]
    b2 = (d2_b * bn2_g * inv + bn2_b)[None, :]
    d3 = d3_k * (bn3_g * inv)[None, :]
    b3 = (d3_b * bn3_g * inv + bn3_b)[None, :]

    w1 = fc_k[0:F, :]
    w2 = fc_k[F:F + E, :]
    w3 = fc_k[F + E:, :]
    bias = fc_b[None, :]

    s = (lax.broadcasted_iota(i32, (FE, E), 0) % E ==
         lax.broadcasted_iota(i32, (FE, E), 1)).astype(f32)

    grid = (B // BLK,)
    bspec = lambda w: pl.BlockSpec((BLK, w), lambda i: (i, 0))
    wspec = lambda shp: pl.BlockSpec(shp, lambda i: (0, 0))

    out = pl.pallas_call(
        _mlp_body,
        grid=grid,
        in_specs=[
            bspec(FE), bspec(F), bspec(F),
            wspec((F, FE)),
            wspec((FE, H)), wspec((1, H)),
            wspec((H, H)), wspec((1, H)),
            wspec((H, H)), wspec((1, H)),
            wspec((FE, E)),
            wspec((F, 1)), wspec((E, 1)), wspec((H, 1)), wspec((1, 1)),
        ],
        out_specs=pl.BlockSpec((BLK, 1), lambda i: (i, 0)),
        out_shape=jax.ShapeDtypeStruct((B, 1), f32),
    )(fev_raw, yfw_raw, fv, r26, d1, b1, d2, b2, d3, b3, s, w1, w2, w3, bias)
    return out
